# baseline (speedup 1.0000x reference)
import functools
import math

import jax
import jax.numpy as jnp
from jax import lax
from jax.experimental import pallas as pl
from jax.experimental.pallas import tpu as pltpu
from jax.experimental.pallas import tpu_sc as plsc

QN = 20
MC = 8
NLAYER = 4
H = 4
NA = 1000
AED = 32
F = AED + MC + 384
P = MC
DFF = 4 * F
DH = F // H

FP = 512
DHP = 128
DFFP = 1792
HLF = F // 2
HLFP = 256

BB = 4


def _sc_gather(q_emb, a_emb, qidx, aidx):
    n = qidx.shape[0]
    d = q_emb.shape[1]
    info = plsc.get_sparse_core_info()
    nc, ns = info.num_cores, info.num_subcores
    nw = nc * ns
    per_w = n // nw
    mesh = plsc.VectorSubcoreMesh(core_axis_name="c", subcore_axis_name="s")

    @functools.partial(
        pl.kernel,
        mesh=mesh,
        out_type=[
            jax.ShapeDtypeStruct((n, d), jnp.float32),
            jax.ShapeDtypeStruct((n, d), jnp.float32),
        ],
        scratch_types=[
            pltpu.VMEM((per_w,), jnp.int32),
            pltpu.VMEM((per_w,), jnp.int32),
            pltpu.VMEM((per_w, d), jnp.float32),
            pltpu.VMEM((per_w, d), jnp.float32),
            pltpu.SemaphoreType.DMA,
            pltpu.SemaphoreType.DMA,
        ],
    )
    def gather_k(qt_hbm, at_hbm, qi_hbm, ai_hbm, qo_hbm, ao_hbm,
                 qi_v, ai_v, qr_v, ar_v, sem_q, sem_a):
        wid = lax.axis_index("s") * nc + lax.axis_index("c")
        base = wid * per_w
        pltpu.sync_copy(qi_hbm.at[pl.ds(base, per_w)], qi_v)
        pltpu.sync_copy(ai_hbm.at[pl.ds(base, per_w)], ai_v)
        cq = pltpu.async_copy(qt_hbm.at[qi_v], qr_v, sem_q)
        ca = pltpu.async_copy(at_hbm.at[ai_v], ar_v, sem_a)
        cq.wait()
        ca.wait()
        pltpu.sync_copy(qr_v, qo_hbm.at[pl.ds(base, per_w)])
        pltpu.sync_copy(ar_v, ao_hbm.at[pl.ds(base, per_w)])

    return gather_k(q_emb, a_emb, qidx, aidx)


RAW_KEYS = ("Qw", "Qb", "Kw", "Kb", "Vw", "Vb", "Ow", "Ob",
            "ff1w", "ff1b", "ff2w", "ff2b",
            "n1a", "n1b", "n2a", "n2b",
            "puw", "pub",
            "s1w", "s1b", "s2w", "s2b",
            "c1w", "c1b", "c2w", "c2b")


def _bf(x):
    return x.astype(jnp.bfloat16)


def _pad_rc(w, rows, cols):
    r, c = w.shape
    if cols > c:
        w = jnp.concatenate([w, jnp.zeros((r, cols - c), w.dtype)], axis=1)
    if rows > r:
        w = jnp.concatenate([w, jnp.zeros((rows - r, cols), w.dtype)], axis=0)
    return w


def _perm_cols(w):
    z = jnp.zeros((w.shape[0], DHP - DH), w.dtype)
    parts = []
    for h in range(H):
        parts.append(w[:, h * DH:(h + 1) * DH])
        parts.append(z)
    return jnp.concatenate(parts, axis=1)


def _perm_rows(w):
    z = jnp.zeros((DHP - DH, w.shape[1]), w.dtype)
    parts = []
    for h in range(H):
        parts.append(w[h * DH:(h + 1) * DH, :])
        parts.append(z)
    return jnp.concatenate(parts, axis=0)


def _ln(y, a, b, fmask):
    m = jnp.sum(y, axis=-1, keepdims=True) * (1.0 / F)
    c = y - m
    var = jnp.sum(c * c * fmask, axis=-1, keepdims=True) * (1.0 / (F - 1))
    return a * (c / (jnp.sqrt(var) + 1e-6)) + b


def _layer_body(args, write_fx):
    (fx_ref, px_ref, qrow_ref, qcol_ref,
     rQw, rQb, rKw, rKb, rVw, rVb, rOw, rOb,
     rf1w, rf1b, rf2w, rf2b,
     rn1a, rn1b, rn2a, rn2b,
     rpuw, rpub,
     rs1w, rs1b, rs2w, rs2b,
     rc1w, rc1b, rc2w, rc2b) = args[:30]
    outs = args[30:30 + (2 if write_fx else 1)]
    (qw_s, qb_s, kw_s, kb_s, vw_s, vb_s, ow_s, ob_s,
     f1w_s, f1b_s, f2w_s, f2b_s,
     n1a_s, n1b_s, n2a_s, n2b_s,
     pf_s, pp_s,
     s1w_s, s1b_s, s2w_s,
     c1w_s, c1b_s, c2w_s) = args[30 + len(outs):]
    if write_fx:
        fx_out, px_out = outs
    else:
        fx_out, (px_out,) = None, outs

    @pl.when(pl.program_id(0) == 0)
    def _prep():
        qw_s[...] = _bf(_pad_rc(_perm_cols(rQw[...]), FP, FP))
        kw_s[...] = _bf(_pad_rc(_perm_cols(rKw[...]), FP, FP))
        vw_s[...] = _bf(_pad_rc(_perm_cols(rVw[...]), FP, FP))
        ow_s[...] = _bf(_pad_rc(_perm_rows(rOw[...]), FP, FP))
        qb_s[...] = _perm_cols(rQb[...].reshape(1, F))
        kb_s[...] = _perm_cols(rKb[...].reshape(1, F))
        vb_s[...] = _perm_cols(rVb[...].reshape(1, F))
        ob_s[...] = _pad_rc(rOb[...].reshape(1, F), 1, FP)
        f1w_s[...] = _bf(_pad_rc(rf1w[...], FP, DFFP))
        f1b_s[...] = _pad_rc(rf1b[...].reshape(1, DFF), 1, DFFP)
        f2w_s[...] = _bf(_pad_rc(rf2w[...], DFFP, FP))
        f2b_s[...] = _pad_rc(rf2b[...].reshape(1, F), 1, FP)
        n1a_s[...] = _pad_rc(rn1a[...].reshape(1, F), 1, FP)
        n1b_s[...] = _pad_rc(rn1b[...].reshape(1, F), 1, FP)
        n2a_s[...] = _pad_rc(rn2a[...].reshape(1, F), 1, FP)
        n2b_s[...] = _pad_rc(rn2b[...].reshape(1, F), 1, FP)
        pf_s[...] = _bf(_pad_rc(rpuw[...][:F, :], FP, P))
        pp_s[...] = _bf(rpuw[...][F:, :])
        s1w_s[...] = _bf(_pad_rc(rs1w[...], FP, HLFP))
        s1b_s[...] = _pad_rc(rs1b[...].reshape(1, HLF), 1, HLFP)
        s2w_s[...] = _pad_rc(rs2w[...].reshape(1, HLF), 1, HLFP)
        c2w_s[...] = _pad_rc(rc2w[...].reshape(1, HLF), 1, HLFP)
        c1w_s[...] = _bf(_pad_rc(rc1w[...], FP, HLFP))
        c1b_s[...] = _pad_rc(rc1b[...].reshape(1, HLF), 1, HLFP)

    s = fx_ref.shape[1]
    fx = fx_ref[...].reshape(BB * s, FP)
    px = px_ref[...].reshape(BB * s, P)
    fmask = (lax.broadcasted_iota(jnp.int32, (1, FP), 1) < F
             ).astype(jnp.float32)

    fxb = _bf(fx)
    q = jnp.dot(fxb, qw_s[...], preferred_element_type=jnp.float32) + qb_s[...]
    k = jnp.dot(fxb, kw_s[...], preferred_element_type=jnp.float32) + kb_s[...]
    v = jnp.dot(fxb, vw_s[...], preferred_element_type=jnp.float32) + vb_s[...]

    qb16, kb16, vb16 = _bf(q), _bf(k), _bf(v)
    scale = 1.0 / math.sqrt(DH)
    ctx_rows = []
    for i in range(BB):
        rs = slice(i * s, (i + 1) * s)
        ctxs = []
        for h in range(H):
            sl = slice(h * DHP, (h + 1) * DHP)
            sc = lax.dot_general(qb16[rs, sl], kb16[rs, sl],
                                 (((1,), (1,)), ((), ())),
                                 preferred_element_type=jnp.float32) * scale
            sc = sc - jnp.max(sc, axis=-1, keepdims=True)
            e = jnp.exp(sc)
            p = e / jnp.sum(e, axis=-1, keepdims=True)
            ctxs.append(jnp.dot(_bf(p), vb16[rs, sl],
                                preferred_element_type=jnp.float32))
        ctx_rows.append(jnp.concatenate(ctxs, axis=-1))
    ctx = _bf(jnp.concatenate(ctx_rows, axis=0))
    att = jnp.dot(ctx, ow_s[...], preferred_element_type=jnp.float32) + ob_s[...]

    fx1 = _ln(fx + att, n1a_s[...], n1b_s[...], fmask)

    ff = jnp.maximum(
        jnp.dot(_bf(fx1), f1w_s[...], preferred_element_type=jnp.float32)
        + f1b_s[...], 0.0)
    ff = jnp.dot(_bf(ff), f2w_s[...],
                 preferred_element_type=jnp.float32) + f2b_s[...]
    fx2 = _ln(fx1 + ff, n2a_s[...], n2b_s[...], fmask)

    fx2b = _bf(fx2)
    px_new = (jnp.dot(fx2b, pf_s[...], preferred_element_type=jnp.float32)
              + jnp.dot(_bf(px), pp_s[...], preferred_element_type=jnp.float32)
              + rpub[...].reshape(1, P))

    h1 = jnp.maximum(
        jnp.dot(fx2b, s1w_s[...], preferred_element_type=jnp.float32)
        + s1b_s[...], 0.0)
    sim = (jnp.sum(h1 * s2w_s[...], axis=-1, keepdims=True)
           + rs2b[...].reshape(1, 1))
    h2 = jnp.maximum(
        jnp.dot(fx2b, c1w_s[...], preferred_element_type=jnp.float32)
        + c1b_s[...], 0.0)
    conf = jax.nn.sigmoid(jnp.sum(h2 * c2w_s[...], axis=-1, keepdims=True)
                          + rc2b[...].reshape(1, 1))

    sm_rows = []
    for i in range(BB):
        rs = slice(i * s, (i + 1) * s)
        qrow = qrow_ref[i]
        qcol = qcol_ref[i]
        qmask = (qcol == qrow).astype(jnp.float32)
        m = sim[rs] * qmask
        m = m - jnp.max(m, axis=0, keepdims=True)
        e = jnp.exp(m)
        aw = e / jnp.sum(e, axis=0, keepdims=True)
        sm_rows.append(
            lax.dot_general(_bf(aw), _bf(px_new[rs]), (((0,), (0,)), ((), ())),
                            preferred_element_type=jnp.float32))
    smoothed = jnp.concatenate(sm_rows, axis=0)

    if fx_out is not None:
        fx_out[...] = fx2.reshape(BB, s, FP)
    px_out[...] = (conf * px_new
                   + (1.0 - conf) * smoothed).reshape(BB, s, P)


def _body_mid(*args):
    _layer_body(args, write_fx=True)


def _body_last(*args):
    _layer_body(args, write_fx=False)


def _whole(shape):
    nd = len(shape)
    return pl.BlockSpec(shape, lambda b, _nd=nd: (0,) * _nd)


_SCRATCH = [
    pltpu.VMEM((FP, FP), jnp.bfloat16),
    pltpu.VMEM((1, FP), jnp.float32),
    pltpu.VMEM((FP, FP), jnp.bfloat16),
    pltpu.VMEM((1, FP), jnp.float32),
    pltpu.VMEM((FP, FP), jnp.bfloat16),
    pltpu.VMEM((1, FP), jnp.float32),
    pltpu.VMEM((FP, FP), jnp.bfloat16),
    pltpu.VMEM((1, FP), jnp.float32),
    pltpu.VMEM((FP, DFFP), jnp.bfloat16),
    pltpu.VMEM((1, DFFP), jnp.float32),
    pltpu.VMEM((DFFP, FP), jnp.bfloat16),
    pltpu.VMEM((1, FP), jnp.float32),
    pltpu.VMEM((1, FP), jnp.float32),
    pltpu.VMEM((1, FP), jnp.float32),
    pltpu.VMEM((1, FP), jnp.float32),
    pltpu.VMEM((1, FP), jnp.float32),
    pltpu.VMEM((FP, P), jnp.bfloat16),
    pltpu.VMEM((P, P), jnp.bfloat16),
    pltpu.VMEM((FP, HLFP), jnp.bfloat16),
    pltpu.VMEM((1, HLFP), jnp.float32),
    pltpu.VMEM((1, HLFP), jnp.float32),
    pltpu.VMEM((FP, HLFP), jnp.bfloat16),
    pltpu.VMEM((1, HLFP), jnp.float32),
    pltpu.VMEM((1, HLFP), jnp.float32),
]


def _layer_call(fx, px, qrow, qcol, wts, last=False, interpret=False):
    b, s, _ = fx.shape
    in_specs = [
        pl.BlockSpec((BB, s, FP), lambda i: (i, 0, 0)),
        pl.BlockSpec((BB, s, P), lambda i: (i, 0, 0)),
        pl.BlockSpec((BB, 1, s), lambda i: (i, 0, 0)),
        pl.BlockSpec((BB, s, 1), lambda i: (i, 0, 0)),
    ] + [_whole(w.shape) for w in wts]
    px_spec = pl.BlockSpec((BB, s, P), lambda i: (i, 0, 0))
    px_shape = jax.ShapeDtypeStruct((b, s, P), jnp.float32)
    if last:
        out_specs, out_shape = px_spec, px_shape
        body = _body_last
    else:
        out_specs = [pl.BlockSpec((BB, s, FP), lambda i: (i, 0, 0)), px_spec]
        out_shape = [jax.ShapeDtypeStruct((b, s, FP), jnp.float32), px_shape]
        body = _body_mid
    out = pl.pallas_call(
        body,
        grid=(b // BB,),
        in_specs=in_specs,
        out_specs=out_specs,
        out_shape=out_shape,
        scratch_shapes=list(_SCRATCH),
        compiler_params=pltpu.CompilerParams(
            dimension_semantics=("arbitrary",),
        ),
        interpret=interpret,
    )(fx, px, qrow, qcol, *wts)
    if last:
        return None, out
    return out


def kernel(x, annotators, questions, embeddings, params):
    b, s = annotators.shape
    qidx = questions.astype(jnp.int32)
    ann = annotators.astype(jnp.int32)
    aidx = jnp.where(ann < 0, NA, ann)

    qt = jnp.pad(params["q_emb"], ((0, 0), (0, 128 - AED)))
    at = jnp.pad(params["a_emb"], ((0, 0), (0, 128 - AED)))
    qe, ae = _sc_gather(qt, at, qidx.reshape(-1), aidx.reshape(-1))
    emb = (qe + ae)[:, :AED].reshape(b, s, AED)

    fx = jnp.concatenate(
        [emb, embeddings, x[:, :, 1:],
         jnp.zeros((b, s, FP - F), jnp.float32)], axis=-1)
    px = x[:, :, 1:]
    qrow = qidx.reshape(b, 1, s)
    qcol = qidx.reshape(b, s, 1)

    for li, lp in enumerate(params["layers"]):
        wts = [lp[k][:, 0] if k in ("s2w", "c2w") else lp[k]
               for k in RAW_KEYS]
        fx, px = _layer_call(fx, px, qrow, qcol, wts,
                             last=(li == NLAYER - 1))
    return px


# speedup vs baseline: 1.2034x; 1.2034x over previous
import functools
import math

import jax
import jax.numpy as jnp
from jax import lax
from jax.experimental import pallas as pl
from jax.experimental.pallas import tpu as pltpu
from jax.experimental.pallas import tpu_sc as plsc

QN = 20
MC = 8
NLAYER = 4
H = 4
NA = 1000
AED = 32
F = AED + MC + 384
P = MC
DFF = 4 * F
DH = F // H

FP = 512
DHP = 128
DFFP = 1792
HLF = F // 2
HLFP = 256

BB = 2


def _sc_gather(a_emb, aidx):
    n = aidx.shape[0]
    d = a_emb.shape[1]
    info = plsc.get_sparse_core_info()
    nc, ns = info.num_cores, info.num_subcores
    nw = nc * ns
    per_w = n // nw
    mesh = plsc.VectorSubcoreMesh(core_axis_name="c", subcore_axis_name="s")

    @functools.partial(
        pl.kernel,
        mesh=mesh,
        out_type=jax.ShapeDtypeStruct((n, d), jnp.float32),
        scratch_types=[
            pltpu.VMEM((per_w,), jnp.int32),
            pltpu.VMEM((per_w, d), jnp.float32),
            pltpu.SemaphoreType.DMA,
        ],
    )
    def gather_k(at_hbm, ai_hbm, ao_hbm, ai_v, ar_v, sem_a):
        wid = lax.axis_index("s") * nc + lax.axis_index("c")
        base = wid * per_w
        pltpu.sync_copy(ai_hbm.at[pl.ds(base, per_w)], ai_v)
        pltpu.async_copy(at_hbm.at[ai_v], ar_v, sem_a).wait()
        pltpu.sync_copy(ar_v, ao_hbm.at[pl.ds(base, per_w)])

    return gather_k(a_emb, aidx)


RAW_KEYS = ("Qw", "Qb", "Kw", "Kb", "Vw", "Vb", "Ow", "Ob",
            "ff1w", "ff1b", "ff2w", "ff2b",
            "n1a", "n1b", "n2a", "n2b",
            "puw", "pub",
            "s1w", "s1b", "s2w", "s2b",
            "c1w", "c1b", "c2w", "c2b")


def _bf(x):
    return x.astype(jnp.bfloat16)


def _pad_rc(w, rows, cols):
    r, c = w.shape
    if cols > c:
        w = jnp.concatenate([w, jnp.zeros((r, cols - c), w.dtype)], axis=1)
    if rows > r:
        w = jnp.concatenate([w, jnp.zeros((rows - r, cols), w.dtype)], axis=0)
    return w


def _perm_cols(w):
    z = jnp.zeros((w.shape[0], DHP - DH), w.dtype)
    parts = []
    for h in range(H):
        parts.append(w[:, h * DH:(h + 1) * DH])
        parts.append(z)
    return jnp.concatenate(parts, axis=1)


def _perm_rows(w):
    z = jnp.zeros((DHP - DH, w.shape[1]), w.dtype)
    parts = []
    for h in range(H):
        parts.append(w[h * DH:(h + 1) * DH, :])
        parts.append(z)
    return jnp.concatenate(parts, axis=0)


def _ln(y, a, b, fmask):
    m = jnp.sum(y, axis=-1, keepdims=True) * (1.0 / F)
    c = y - m
    var = jnp.sum(c * c * fmask, axis=-1, keepdims=True) * (1.0 / (F - 1))
    return a * (c / (jnp.sqrt(var) + 1e-6)) + b


def _layer_body(args, write_fx, first=False):
    if first:
        (x_ref, ae_ref, emb_ref, qemb_ref, qrow_ref, qcol_ref) = args[:6]
        nin = 6 + 26
        raws = args[6:nin]
    else:
        (fx_ref, px_ref, qrow_ref, qcol_ref) = args[:4]
        nin = 4 + 26
        raws = args[4:nin]
    (rQw, rQb, rKw, rKb, rVw, rVb, rOw, rOb,
     rf1w, rf1b, rf2w, rf2b,
     rn1a, rn1b, rn2a, rn2b,
     rpuw, rpub,
     rs1w, rs1b, rs2w, rs2b,
     rc1w, rc1b, rc2w, rc2b) = raws
    outs = args[nin:nin + (2 if write_fx else 1)]
    (qw_s, qb_s, kw_s, kb_s, vw_s, vb_s, ow_s, ob_s,
     f1w_s, f1b_s, f2w_s, f2b_s,
     n1a_s, n1b_s, n2a_s, n2b_s,
     pf_s, pp_s,
     s1w_s, s1b_s, s2w_s,
     c1w_s, c1b_s, c2w_s) = args[nin + len(outs):]
    if write_fx:
        fx_out, px_out = outs
    else:
        fx_out, (px_out,) = None, outs

    @pl.when(pl.program_id(0) == 0)
    def _prep():
        qw_s[...] = _bf(_pad_rc(_perm_cols(rQw[...]), FP, FP))
        kw_s[...] = _bf(_pad_rc(_perm_cols(rKw[...]), FP, FP))
        vw_s[...] = _bf(_pad_rc(_perm_cols(rVw[...]), FP, FP))
        ow_s[...] = _bf(_pad_rc(_perm_rows(rOw[...]), FP, FP))
        qb_s[...] = _perm_cols(rQb[...].reshape(1, F))
        kb_s[...] = _perm_cols(rKb[...].reshape(1, F))
        vb_s[...] = _perm_cols(rVb[...].reshape(1, F))
        ob_s[...] = _pad_rc(rOb[...].reshape(1, F), 1, FP)
        f1w_s[...] = _bf(_pad_rc(rf1w[...], FP, DFFP))
        f1b_s[...] = _pad_rc(rf1b[...].reshape(1, DFF), 1, DFFP)
        f2w_s[...] = _bf(_pad_rc(rf2w[...], DFFP, FP))
        f2b_s[...] = _pad_rc(rf2b[...].reshape(1, F), 1, FP)
        n1a_s[...] = _pad_rc(rn1a[...].reshape(1, F), 1, FP)
        n1b_s[...] = _pad_rc(rn1b[...].reshape(1, F), 1, FP)
        n2a_s[...] = _pad_rc(rn2a[...].reshape(1, F), 1, FP)
        n2b_s[...] = _pad_rc(rn2b[...].reshape(1, F), 1, FP)
        pf_s[...] = _bf(_pad_rc(rpuw[...][:F, :], FP, P))
        pp_s[...] = _bf(rpuw[...][F:, :])
        s1w_s[...] = _bf(_pad_rc(rs1w[...], FP, HLFP))
        s1b_s[...] = _pad_rc(rs1b[...].reshape(1, HLF), 1, HLFP)
        s2w_s[...] = _pad_rc(rs2w[...].reshape(1, HLF), 1, HLFP)
        c2w_s[...] = _pad_rc(rc2w[...].reshape(1, HLF), 1, HLFP)
        c1w_s[...] = _bf(_pad_rc(rc1w[...], FP, HLFP))
        c1b_s[...] = _pad_rc(rc1b[...].reshape(1, HLF), 1, HLFP)

    if first:
        s = x_ref.shape[1]
        x9 = x_ref[...].reshape(BB * s, MC + 1)
        px = x9[:, 1:]
        ae = ae_ref[...].reshape(BB * s, 128)[:, :AED]
        emb = emb_ref[...].reshape(BB * s, 384)
        qc = qcol_ref[...].reshape(BB * s, 1)
        oh = (qc == lax.broadcasted_iota(jnp.int32, (1, QN), 1)
              ).astype(jnp.float32)
        qe = jnp.dot(oh, qemb_ref[...], preferred_element_type=jnp.float32)
        fx = jnp.concatenate(
            [ae + qe, emb, px,
             jnp.zeros((BB * s, FP - F), jnp.float32)], axis=1)
    else:
        s = fx_ref.shape[1]
        fx = fx_ref[...].reshape(BB * s, FP)
        px = px_ref[...].reshape(BB * s, P)
    fmask = (lax.broadcasted_iota(jnp.int32, (1, FP), 1) < F
             ).astype(jnp.float32)

    fxb = _bf(fx)
    q = jnp.dot(fxb, qw_s[...], preferred_element_type=jnp.float32) + qb_s[...]
    k = jnp.dot(fxb, kw_s[...], preferred_element_type=jnp.float32) + kb_s[...]
    v = jnp.dot(fxb, vw_s[...], preferred_element_type=jnp.float32) + vb_s[...]

    qb16, kb16, vb16 = _bf(q), _bf(k), _bf(v)
    scale = 1.0 / math.sqrt(DH)
    ctx_rows = []
    for i in range(BB):
        rs = slice(i * s, (i + 1) * s)
        ctxs = []
        for h in range(H):
            sl = slice(h * DHP, (h + 1) * DHP)
            sc = lax.dot_general(qb16[rs, sl], kb16[rs, sl],
                                 (((1,), (1,)), ((), ())),
                                 preferred_element_type=jnp.float32) * scale
            sc = sc - jnp.max(sc, axis=-1, keepdims=True)
            e = jnp.exp(sc)
            p = e / jnp.sum(e, axis=-1, keepdims=True)
            ctxs.append(jnp.dot(_bf(p), vb16[rs, sl],
                                preferred_element_type=jnp.float32))
        ctx_rows.append(jnp.concatenate(ctxs, axis=-1))
    ctx = _bf(jnp.concatenate(ctx_rows, axis=0))
    att = jnp.dot(ctx, ow_s[...], preferred_element_type=jnp.float32) + ob_s[...]

    fx1 = _ln(fx + att, n1a_s[...], n1b_s[...], fmask)

    ff = jnp.maximum(
        jnp.dot(_bf(fx1), f1w_s[...], preferred_element_type=jnp.float32)
        + f1b_s[...], 0.0)
    ff = jnp.dot(_bf(ff), f2w_s[...],
                 preferred_element_type=jnp.float32) + f2b_s[...]
    fx2 = _ln(fx1 + ff, n2a_s[...], n2b_s[...], fmask)

    fx2b = _bf(fx2)
    px_new = (jnp.dot(fx2b, pf_s[...], preferred_element_type=jnp.float32)
              + jnp.dot(_bf(px), pp_s[...], preferred_element_type=jnp.float32)
              + rpub[...].reshape(1, P))

    h1 = jnp.maximum(
        jnp.dot(fx2b, s1w_s[...], preferred_element_type=jnp.float32)
        + s1b_s[...], 0.0)
    sim = (jnp.sum(h1 * s2w_s[...], axis=-1, keepdims=True)
           + rs2b[...].reshape(1, 1))
    h2 = jnp.maximum(
        jnp.dot(fx2b, c1w_s[...], preferred_element_type=jnp.float32)
        + c1b_s[...], 0.0)
    conf = jax.nn.sigmoid(jnp.sum(h2 * c2w_s[...], axis=-1, keepdims=True)
                          + rc2b[...].reshape(1, 1))

    sm_rows = []
    for i in range(BB):
        rs = slice(i * s, (i + 1) * s)
        qrow = qrow_ref[i]
        qcol = qcol_ref[i]
        qmask = (qcol == qrow).astype(jnp.float32)
        m = sim[rs] * qmask
        m = m - jnp.max(m, axis=0, keepdims=True)
        e = jnp.exp(m)
        aw = e / jnp.sum(e, axis=0, keepdims=True)
        sm_rows.append(
            lax.dot_general(_bf(aw), _bf(px_new[rs]), (((0,), (0,)), ((), ())),
                            preferred_element_type=jnp.float32))
    smoothed = jnp.concatenate(sm_rows, axis=0)

    if fx_out is not None:
        fx_out[...] = fx2.reshape(BB, s, FP)
    px_out[...] = (conf * px_new
                   + (1.0 - conf) * smoothed).reshape(BB, s, P)


def _body_mid(*args):
    _layer_body(args, write_fx=True)


def _body_last(*args):
    _layer_body(args, write_fx=False)


def _body_first(*args):
    _layer_body(args, write_fx=True, first=True)


def _whole(shape):
    nd = len(shape)
    return pl.BlockSpec(shape, lambda b, _nd=nd: (0,) * _nd)


_SCRATCH = [
    pltpu.VMEM((FP, FP), jnp.bfloat16),
    pltpu.VMEM((1, FP), jnp.float32),
    pltpu.VMEM((FP, FP), jnp.bfloat16),
    pltpu.VMEM((1, FP), jnp.float32),
    pltpu.VMEM((FP, FP), jnp.bfloat16),
    pltpu.VMEM((1, FP), jnp.float32),
    pltpu.VMEM((FP, FP), jnp.bfloat16),
    pltpu.VMEM((1, FP), jnp.float32),
    pltpu.VMEM((FP, DFFP), jnp.bfloat16),
    pltpu.VMEM((1, DFFP), jnp.float32),
    pltpu.VMEM((DFFP, FP), jnp.bfloat16),
    pltpu.VMEM((1, FP), jnp.float32),
    pltpu.VMEM((1, FP), jnp.float32),
    pltpu.VMEM((1, FP), jnp.float32),
    pltpu.VMEM((1, FP), jnp.float32),
    pltpu.VMEM((1, FP), jnp.float32),
    pltpu.VMEM((FP, P), jnp.bfloat16),
    pltpu.VMEM((P, P), jnp.bfloat16),
    pltpu.VMEM((FP, HLFP), jnp.bfloat16),
    pltpu.VMEM((1, HLFP), jnp.float32),
    pltpu.VMEM((1, HLFP), jnp.float32),
    pltpu.VMEM((FP, HLFP), jnp.bfloat16),
    pltpu.VMEM((1, HLFP), jnp.float32),
    pltpu.VMEM((1, HLFP), jnp.float32),
]


def _layer_call(ins, qrow, qcol, wts, first=False, last=False,
                interpret=False):
    b, _, s = qrow.shape
    if first:
        x, ae, emb, qemb = ins
        in_specs = [
            pl.BlockSpec((BB, s, MC + 1), lambda i: (i, 0, 0)),
            pl.BlockSpec((BB, s, 128), lambda i: (i, 0, 0)),
            pl.BlockSpec((BB, s, 384), lambda i: (i, 0, 0)),
            _whole(qemb.shape),
        ]
        operands = [x, ae, emb, qemb]
        body0 = _body_first
    else:
        fx, px = ins
        in_specs = [
            pl.BlockSpec((BB, s, FP), lambda i: (i, 0, 0)),
            pl.BlockSpec((BB, s, P), lambda i: (i, 0, 0)),
        ]
        operands = [fx, px]
        body0 = _body_mid
    in_specs += [
        pl.BlockSpec((BB, 1, s), lambda i: (i, 0, 0)),
        pl.BlockSpec((BB, s, 1), lambda i: (i, 0, 0)),
    ] + [_whole(w.shape) for w in wts]
    px_spec = pl.BlockSpec((BB, s, P), lambda i: (i, 0, 0))
    px_shape = jax.ShapeDtypeStruct((b, s, P), jnp.float32)
    if last:
        out_specs, out_shape = px_spec, px_shape
        body = _body_last
    else:
        out_specs = [pl.BlockSpec((BB, s, FP), lambda i: (i, 0, 0)), px_spec]
        out_shape = [jax.ShapeDtypeStruct((b, s, FP), jnp.float32), px_shape]
        body = body0
    out = pl.pallas_call(
        body,
        grid=(b // BB,),
        in_specs=in_specs,
        out_specs=out_specs,
        out_shape=out_shape,
        scratch_shapes=list(_SCRATCH),
        compiler_params=pltpu.CompilerParams(
            dimension_semantics=("arbitrary",),
        ),
        interpret=interpret,
    )(*operands, qrow, qcol, *wts)
    if last:
        return None, out
    return out


def kernel(x, annotators, questions, embeddings, params):
    b, s = annotators.shape
    qidx = questions.astype(jnp.int32)
    ann = annotators.astype(jnp.int32)
    aidx = jnp.where(ann < 0, NA, ann)

    at = jnp.pad(params["a_emb"], ((0, 0), (0, 128 - AED)))
    ae = _sc_gather(at, aidx.reshape(-1)).reshape(b, s, 128)

    qrow = qidx.reshape(b, 1, s)
    qcol = qidx.reshape(b, s, 1)

    fx, px = None, None
    for li, lp in enumerate(params["layers"]):
        wts = [lp[k][:, 0] if k in ("s2w", "c2w") else lp[k]
               for k in RAW_KEYS]
        first = li == 0
        ins = (x, ae, embeddings, params["q_emb"]) if first else (fx, px)
        fx, px = _layer_call(ins, qrow, qcol, wts,
                             first=first, last=(li == NLAYER - 1))
    return px


# speedup vs baseline: 1.2695x; 1.0549x over previous
import functools
import math

import jax
import jax.numpy as jnp
from jax import lax
from jax.experimental import pallas as pl
from jax.experimental.pallas import tpu as pltpu
from jax.experimental.pallas import tpu_sc as plsc

QN = 20
MC = 8
NLAYER = 4
H = 4
NA = 1000
AED = 32
F = AED + MC + 384
P = MC
DFF = 4 * F
DH = F // H

FP = 512
DHP = 128
DFFP = 1792
HLF = F // 2
HLFP = 256

BB = 2


def _sc_gather(a_emb, aidx):
    n = aidx.shape[0]
    d = a_emb.shape[1]
    info = plsc.get_sparse_core_info()
    nc, ns = info.num_cores, info.num_subcores
    nw = nc * ns
    per_w = n // nw
    mesh = plsc.VectorSubcoreMesh(core_axis_name="c", subcore_axis_name="s")

    @functools.partial(
        pl.kernel,
        mesh=mesh,
        out_type=jax.ShapeDtypeStruct((n, d), jnp.float32),
        scratch_types=[
            pltpu.VMEM((per_w,), jnp.int32),
            pltpu.VMEM((per_w, d), jnp.float32),
            pltpu.SemaphoreType.DMA,
        ],
    )
    def gather_k(at_hbm, ai_hbm, ao_hbm, ai_v, ar_v, sem_a):
        wid = lax.axis_index("s") * nc + lax.axis_index("c")
        base = wid * per_w
        pltpu.sync_copy(ai_hbm.at[pl.ds(base, per_w)], ai_v)
        pltpu.async_copy(at_hbm.at[ai_v], ar_v, sem_a).wait()
        pltpu.sync_copy(ar_v, ao_hbm.at[pl.ds(base, per_w)])

    return gather_k(a_emb, aidx)


RAW_KEYS = ("Qw", "Qb", "Kw", "Kb", "Vw", "Vb", "Ow", "Ob",
            "ff1w", "ff1b", "ff2w", "ff2b",
            "n1a", "n1b", "n2a", "n2b",
            "puw", "pub",
            "s1w", "s1b", "s2w", "s2b",
            "c1w", "c1b", "c2w", "c2b")


def _bf(x):
    return x.astype(jnp.bfloat16)


def _pad_rc(w, rows, cols):
    r, c = w.shape
    if cols > c:
        w = jnp.concatenate([w, jnp.zeros((r, cols - c), w.dtype)], axis=1)
    if rows > r:
        w = jnp.concatenate([w, jnp.zeros((rows - r, cols), w.dtype)], axis=0)
    return w


def _perm_cols(w):
    z = jnp.zeros((w.shape[0], DHP - DH), w.dtype)
    parts = []
    for h in range(H):
        parts.append(w[:, h * DH:(h + 1) * DH])
        parts.append(z)
    return jnp.concatenate(parts, axis=1)


def _perm_rows(w):
    z = jnp.zeros((DHP - DH, w.shape[1]), w.dtype)
    parts = []
    for h in range(H):
        parts.append(w[h * DH:(h + 1) * DH, :])
        parts.append(z)
    return jnp.concatenate(parts, axis=0)


def _ln(y, a, b):
    m = jnp.sum(y, axis=-1, keepdims=True) * (1.0 / F)
    ssq = jnp.sum(y * y, axis=-1, keepdims=True)
    var = (ssq - F * m * m) * (1.0 / (F - 1))
    return a * ((y - m) / (jnp.sqrt(var) + 1e-6)) + b


def _layer_body(args, write_fx, first=False):
    if first:
        (x_ref, ae_ref, emb_ref, qemb_ref, qrow_ref, qcol_ref) = args[:6]
        nin = 6 + 26
        raws = args[6:nin]
    else:
        (fx_ref, px_ref, qrow_ref, qcol_ref) = args[:4]
        nin = 4 + 26
        raws = args[4:nin]
    (rQw, rQb, rKw, rKb, rVw, rVb, rOw, rOb,
     rf1w, rf1b, rf2w, rf2b,
     rn1a, rn1b, rn2a, rn2b,
     rpuw, rpub,
     rs1w, rs1b, rs2w, rs2b,
     rc1w, rc1b, rc2w, rc2b) = raws
    outs = args[nin:nin + (2 if write_fx else 1)]
    (qkvw_s, qkvb_s, ow_s, ob_s,
     f1w_s, f1b_s, f2w_s, f2b_s,
     n1a_s, n1b_s, n2a_s, n2b_s,
     pf_s, pp_s,
     scw_s, scb_s, sc2_s) = args[nin + len(outs):]
    if write_fx:
        fx_out, px_out = outs
    else:
        fx_out, (px_out,) = None, outs

    @pl.when(pl.program_id(0) == 0)
    def _prep():
        qkvw_s[...] = _bf(jnp.concatenate(
            [_pad_rc(_perm_cols(rQw[...]), FP, FP),
             _pad_rc(_perm_cols(rKw[...]), FP, FP),
             _pad_rc(_perm_cols(rVw[...]), FP, FP)], axis=1))
        qkvb_s[...] = jnp.concatenate(
            [_perm_cols(rQb[...].reshape(1, F)),
             _perm_cols(rKb[...].reshape(1, F)),
             _perm_cols(rVb[...].reshape(1, F))], axis=1)
        ow_s[...] = _bf(_pad_rc(_perm_rows(rOw[...]), FP, FP))
        ob_s[...] = _pad_rc(rOb[...].reshape(1, F), 1, FP)
        f1w_s[...] = _bf(_pad_rc(rf1w[...], FP, DFFP))
        f1b_s[...] = _pad_rc(rf1b[...].reshape(1, DFF), 1, DFFP)
        f2w_s[...] = _bf(_pad_rc(rf2w[...], DFFP, FP))
        f2b_s[...] = _pad_rc(rf2b[...].reshape(1, F), 1, FP)
        n1a_s[...] = _pad_rc(rn1a[...].reshape(1, F), 1, FP)
        n1b_s[...] = _pad_rc(rn1b[...].reshape(1, F), 1, FP)
        n2a_s[...] = _pad_rc(rn2a[...].reshape(1, F), 1, FP)
        n2b_s[...] = _pad_rc(rn2b[...].reshape(1, F), 1, FP)
        pf_s[...] = _bf(_pad_rc(rpuw[...][:F, :], FP, P))
        pp_s[...] = _bf(rpuw[...][F:, :])
        scw_s[...] = _bf(jnp.concatenate(
            [_pad_rc(rs1w[...], FP, HLFP),
             _pad_rc(rc1w[...], FP, HLFP)], axis=1))
        scb_s[...] = jnp.concatenate(
            [_pad_rc(rs1b[...].reshape(1, HLF), 1, HLFP),
             _pad_rc(rc1b[...].reshape(1, HLF), 1, HLFP)], axis=1)
        sc2_s[...] = jnp.concatenate(
            [_pad_rc(rs2w[...].reshape(1, HLF), 1, HLFP),
             _pad_rc(rc2w[...].reshape(1, HLF), 1, HLFP)], axis=1)

    if first:
        s = x_ref.shape[1]
        x9 = x_ref[...].reshape(BB * s, MC + 1)
        px = x9[:, 1:]
        ae = ae_ref[...].reshape(BB * s, 128)[:, :AED]
        emb = emb_ref[...].reshape(BB * s, 384)
        qc = qcol_ref[...].reshape(BB * s, 1)
        oh = (qc == lax.broadcasted_iota(jnp.int32, (1, QN), 1)
              ).astype(jnp.float32)
        qe = jnp.dot(oh, qemb_ref[...], preferred_element_type=jnp.float32)
        fx = jnp.concatenate(
            [ae + qe, emb, px,
             jnp.zeros((BB * s, FP - F), jnp.float32)], axis=1)
    else:
        s = fx_ref.shape[1]
        fx = fx_ref[...].reshape(BB * s, FP)
        px = px_ref[...].reshape(BB * s, P)
    fxb = _bf(fx)
    qkv = (jnp.dot(fxb, qkvw_s[...], preferred_element_type=jnp.float32)
           + qkvb_s[...])
    qkvb16 = _bf(qkv)

    scale = 1.0 / math.sqrt(DH)
    ctx_rows = []
    for i in range(BB):
        rs = slice(i * s, (i + 1) * s)
        ctxs = []
        for h in range(H):
            qsl = slice(h * DHP, (h + 1) * DHP)
            ksl = slice(FP + h * DHP, FP + (h + 1) * DHP)
            vsl = slice(2 * FP + h * DHP, 2 * FP + (h + 1) * DHP)
            sc = lax.dot_general(qkvb16[rs, qsl], qkvb16[rs, ksl],
                                 (((1,), (1,)), ((), ())),
                                 preferred_element_type=jnp.float32) * scale
            e = jnp.exp(sc)
            p = e / jnp.sum(e, axis=-1, keepdims=True)
            ctxs.append(jnp.dot(_bf(p), qkvb16[rs, vsl],
                                preferred_element_type=jnp.float32))
        ctx_rows.append(jnp.concatenate(ctxs, axis=-1))
    ctx = _bf(jnp.concatenate(ctx_rows, axis=0))
    att = jnp.dot(ctx, ow_s[...], preferred_element_type=jnp.float32) + ob_s[...]

    fx1 = _ln(fx + att, n1a_s[...], n1b_s[...])

    ff = jnp.maximum(
        jnp.dot(_bf(fx1), f1w_s[...], preferred_element_type=jnp.float32)
        + f1b_s[...], 0.0)
    ff = jnp.dot(_bf(ff), f2w_s[...],
                 preferred_element_type=jnp.float32) + f2b_s[...]
    fx2 = _ln(fx1 + ff, n2a_s[...], n2b_s[...])

    fx2b = _bf(fx2)
    px_new = (jnp.dot(fx2b, pf_s[...], preferred_element_type=jnp.float32)
              + jnp.dot(_bf(px), pp_s[...], preferred_element_type=jnp.float32)
              + rpub[...].reshape(1, P))

    hc = jnp.maximum(
        jnp.dot(fx2b, scw_s[...], preferred_element_type=jnp.float32)
        + scb_s[...], 0.0)
    prod = hc * sc2_s[...]
    sim = (jnp.sum(prod[:, :HLFP], axis=-1, keepdims=True)
           + rs2b[...].reshape(1, 1))
    conf = jax.nn.sigmoid(jnp.sum(prod[:, HLFP:], axis=-1, keepdims=True)
                          + rc2b[...].reshape(1, 1))

    sm_rows = []
    for i in range(BB):
        rs = slice(i * s, (i + 1) * s)
        qrow = qrow_ref[i]
        qcol = qcol_ref[i]
        qmask = (qcol == qrow).astype(jnp.float32)
        e = jnp.exp(sim[rs] * qmask)
        aw = e / jnp.sum(e, axis=0, keepdims=True)
        sm_rows.append(
            lax.dot_general(_bf(aw), _bf(px_new[rs]), (((0,), (0,)), ((), ())),
                            preferred_element_type=jnp.float32))
    smoothed = jnp.concatenate(sm_rows, axis=0)

    if fx_out is not None:
        fx_out[...] = fx2.reshape(BB, s, FP)
    px_out[...] = (conf * px_new
                   + (1.0 - conf) * smoothed).reshape(BB, s, P)


def _body_mid(*args):
    _layer_body(args, write_fx=True)


def _body_last(*args):
    _layer_body(args, write_fx=False)


def _body_first(*args):
    _layer_body(args, write_fx=True, first=True)


def _whole(shape):
    nd = len(shape)
    return pl.BlockSpec(shape, lambda b, _nd=nd: (0,) * _nd)


_SCRATCH = [
    pltpu.VMEM((FP, 3 * FP), jnp.bfloat16),
    pltpu.VMEM((1, 3 * FP), jnp.float32),
    pltpu.VMEM((FP, FP), jnp.bfloat16),
    pltpu.VMEM((1, FP), jnp.float32),
    pltpu.VMEM((FP, DFFP), jnp.bfloat16),
    pltpu.VMEM((1, DFFP), jnp.float32),
    pltpu.VMEM((DFFP, FP), jnp.bfloat16),
    pltpu.VMEM((1, FP), jnp.float32),
    pltpu.VMEM((1, FP), jnp.float32),
    pltpu.VMEM((1, FP), jnp.float32),
    pltpu.VMEM((1, FP), jnp.float32),
    pltpu.VMEM((1, FP), jnp.float32),
    pltpu.VMEM((FP, P), jnp.bfloat16),
    pltpu.VMEM((P, P), jnp.bfloat16),
    pltpu.VMEM((FP, 2 * HLFP), jnp.bfloat16),
    pltpu.VMEM((1, 2 * HLFP), jnp.float32),
    pltpu.VMEM((1, 2 * HLFP), jnp.float32),
]


def _layer_call(ins, qrow, qcol, wts, first=False, last=False,
                interpret=False):
    b, _, s = qrow.shape
    if first:
        x, ae, emb, qemb = ins
        in_specs = [
            pl.BlockSpec((BB, s, MC + 1), lambda i: (i, 0, 0)),
            pl.BlockSpec((BB, s, 128), lambda i: (i, 0, 0)),
            pl.BlockSpec((BB, s, 384), lambda i: (i, 0, 0)),
            _whole(qemb.shape),
        ]
        operands = [x, ae, emb, qemb]
        body0 = _body_first
    else:
        fx, px = ins
        in_specs = [
            pl.BlockSpec((BB, s, FP), lambda i: (i, 0, 0)),
            pl.BlockSpec((BB, s, P), lambda i: (i, 0, 0)),
        ]
        operands = [fx, px]
        body0 = _body_mid
    in_specs += [
        pl.BlockSpec((BB, 1, s), lambda i: (i, 0, 0)),
        pl.BlockSpec((BB, s, 1), lambda i: (i, 0, 0)),
    ] + [_whole(w.shape) for w in wts]
    px_spec = pl.BlockSpec((BB, s, P), lambda i: (i, 0, 0))
    px_shape = jax.ShapeDtypeStruct((b, s, P), jnp.float32)
    if last:
        out_specs, out_shape = px_spec, px_shape
        body = _body_last
    else:
        out_specs = [pl.BlockSpec((BB, s, FP), lambda i: (i, 0, 0)), px_spec]
        out_shape = [jax.ShapeDtypeStruct((b, s, FP), jnp.float32), px_shape]
        body = body0
    out = pl.pallas_call(
        body,
        grid=(b // BB,),
        in_specs=in_specs,
        out_specs=out_specs,
        out_shape=out_shape,
        scratch_shapes=list(_SCRATCH),
        compiler_params=pltpu.CompilerParams(
            dimension_semantics=("arbitrary",),
        ),
        interpret=interpret,
    )(*operands, qrow, qcol, *wts)
    if last:
        return None, out
    return out


def kernel(x, annotators, questions, embeddings, params):
    b, s = annotators.shape
    qidx = questions.astype(jnp.int32)
    ann = annotators.astype(jnp.int32)
    aidx = jnp.where(ann < 0, NA, ann)

    at = jnp.pad(params["a_emb"], ((0, 0), (0, 128 - AED)))
    ae = _sc_gather(at, aidx.reshape(-1)).reshape(b, s, 128)

    qrow = qidx.reshape(b, 1, s)
    qcol = qidx.reshape(b, s, 1)

    fx, px = None, None
    for li, lp in enumerate(params["layers"]):
        wts = [lp[k][:, 0] if k in ("s2w", "c2w") else lp[k]
               for k in RAW_KEYS]
        first = li == 0
        ins = (x, ae, embeddings, params["q_emb"]) if first else (fx, px)
        fx, px = _layer_call(ins, qrow, qcol, wts,
                             first=first, last=(li == NLAYER - 1))
    return px


# speedup vs baseline: 1.2727x; 1.0025x over previous
import functools
import math

import jax
import jax.numpy as jnp
from jax import lax
from jax.experimental import pallas as pl
from jax.experimental.pallas import tpu as pltpu
from jax.experimental.pallas import tpu_sc as plsc

QN = 20
MC = 8
NLAYER = 4
H = 4
NA = 1000
AED = 32
F = AED + MC + 384
P = MC
DFF = 4 * F
DH = F // H

FP = 512
DHP = 128
DFFP = 1792
HLF = F // 2
HLFP = 256

BB = 2


def _sc_gather(a_emb, aidx):
    n = aidx.shape[0]
    d = a_emb.shape[1]
    info = plsc.get_sparse_core_info()
    nc, ns = info.num_cores, info.num_subcores
    nw = nc * ns
    per_w = n // nw
    mesh = plsc.VectorSubcoreMesh(core_axis_name="c", subcore_axis_name="s")

    @functools.partial(
        pl.kernel,
        mesh=mesh,
        out_type=jax.ShapeDtypeStruct((n, d), jnp.float32),
        scratch_types=[
            pltpu.VMEM((per_w,), jnp.int32),
            pltpu.VMEM((per_w, d), jnp.float32),
            pltpu.SemaphoreType.DMA,
        ],
    )
    def gather_k(at_hbm, ai_hbm, ao_hbm, ai_v, ar_v, sem_a):
        wid = lax.axis_index("s") * nc + lax.axis_index("c")
        base = wid * per_w
        pltpu.sync_copy(ai_hbm.at[pl.ds(base, per_w)], ai_v)
        pltpu.async_copy(at_hbm.at[ai_v], ar_v, sem_a).wait()
        pltpu.sync_copy(ar_v, ao_hbm.at[pl.ds(base, per_w)])

    return gather_k(a_emb, aidx)


RAW_KEYS = ("Qw", "Qb", "Kw", "Kb", "Vw", "Vb", "Ow", "Ob",
            "ff1w", "ff1b", "ff2w", "ff2b",
            "n1a", "n1b", "n2a", "n2b",
            "puw", "pub",
            "s1w", "s1b", "s2w", "s2b",
            "c1w", "c1b", "c2w", "c2b")


def _bf(x):
    return x.astype(jnp.bfloat16)


def _pad_rc(w, rows, cols):
    r, c = w.shape
    if cols > c:
        w = jnp.concatenate([w, jnp.zeros((r, cols - c), w.dtype)], axis=1)
    if rows > r:
        w = jnp.concatenate([w, jnp.zeros((rows - r, cols), w.dtype)], axis=0)
    return w


def _perm_cols(w):
    z = jnp.zeros((w.shape[0], DHP - DH), w.dtype)
    parts = []
    for h in range(H):
        parts.append(w[:, h * DH:(h + 1) * DH])
        parts.append(z)
    return jnp.concatenate(parts, axis=1)


def _perm_rows(w):
    z = jnp.zeros((DHP - DH, w.shape[1]), w.dtype)
    parts = []
    for h in range(H):
        parts.append(w[h * DH:(h + 1) * DH, :])
        parts.append(z)
    return jnp.concatenate(parts, axis=0)


def _ln(y, a, b):
    m = jnp.sum(y, axis=-1, keepdims=True) * (1.0 / F)
    ssq = jnp.sum(y * y, axis=-1, keepdims=True)
    var = (ssq - F * m * m) * (1.0 / (F - 1))
    return a * ((y - m) / (jnp.sqrt(var) + 1e-6)) + b


def _layer_body(args, write_fx, first=False):
    if first:
        (x_ref, ae_ref, emb_ref, qemb_ref, qrow_ref, qcol_ref) = args[:6]
        nin = 6 + 26
        raws = args[6:nin]
    else:
        (fx_ref, px_ref, qrow_ref, qcol_ref) = args[:4]
        nin = 4 + 26
        raws = args[4:nin]
    (rQw, rQb, rKw, rKb, rVw, rVb, rOw, rOb,
     rf1w, rf1b, rf2w, rf2b,
     rn1a, rn1b, rn2a, rn2b,
     rpuw, rpub,
     rs1w, rs1b, rs2w, rs2b,
     rc1w, rc1b, rc2w, rc2b) = raws
    outs = args[nin:nin + (2 if write_fx else 1)]
    (qkvw_s, qkvb_s, ow_s, ob_s,
     f1w_s, f1b_s, f2w_s, f2b_s,
     n1a_s, n1b_s, n2a_s, n2b_s,
     pf_s, pp_s,
     scw_s, scb_s, sc2_s) = args[nin + len(outs):]
    if write_fx:
        fx_out, px_out = outs
    else:
        fx_out, (px_out,) = None, outs

    scale = 1.0 / math.sqrt(DH)

    @pl.when(pl.program_id(0) == 0)
    def _prep():
        qkvw_s[...] = _bf(jnp.concatenate(
            [_pad_rc(_perm_cols(rQw[...]), FP, FP) * scale,
             _pad_rc(_perm_cols(rKw[...]), FP, FP),
             _pad_rc(_perm_cols(rVw[...]), FP, FP)], axis=1))
        qkvb_s[...] = jnp.concatenate(
            [_perm_cols(rQb[...].reshape(1, F)) * scale,
             _perm_cols(rKb[...].reshape(1, F)),
             _perm_cols(rVb[...].reshape(1, F))], axis=1)
        ow_s[...] = _bf(_pad_rc(_perm_rows(rOw[...]), FP, FP))
        ob_s[...] = _pad_rc(rOb[...].reshape(1, F), 1, FP)
        f1w_s[...] = _bf(_pad_rc(rf1w[...], FP, DFFP))
        f1b_s[...] = _pad_rc(rf1b[...].reshape(1, DFF), 1, DFFP)
        f2w_s[...] = _bf(_pad_rc(rf2w[...], DFFP, FP))
        f2b_s[...] = _pad_rc(rf2b[...].reshape(1, F), 1, FP)
        n1a_s[...] = _pad_rc(rn1a[...].reshape(1, F), 1, FP)
        n1b_s[...] = _pad_rc(rn1b[...].reshape(1, F), 1, FP)
        n2a_s[...] = _pad_rc(rn2a[...].reshape(1, F), 1, FP)
        n2b_s[...] = _pad_rc(rn2b[...].reshape(1, F), 1, FP)
        pf_s[...] = _bf(_pad_rc(rpuw[...][:F, :], FP, P))
        pp_s[...] = _bf(rpuw[...][F:, :])
        scw_s[...] = _bf(jnp.concatenate(
            [_pad_rc(rs1w[...], FP, HLFP),
             _pad_rc(rc1w[...], FP, HLFP)], axis=1))
        scb_s[...] = jnp.concatenate(
            [_pad_rc(rs1b[...].reshape(1, HLF), 1, HLFP),
             _pad_rc(rc1b[...].reshape(1, HLF), 1, HLFP)], axis=1)
        sc2_s[...] = jnp.concatenate(
            [_pad_rc(rs2w[...].reshape(1, HLF), 1, HLFP),
             _pad_rc(rc2w[...].reshape(1, HLF), 1, HLFP)], axis=1)

    if first:
        s = x_ref.shape[1]
        x9 = x_ref[...].reshape(BB * s, MC + 1)
        px = x9[:, 1:]
        ae = ae_ref[...].reshape(BB * s, 128)[:, :AED]
        emb = emb_ref[...].reshape(BB * s, 384)
        qc = qcol_ref[...].reshape(BB * s, 1)
        oh = (qc == lax.broadcasted_iota(jnp.int32, (1, QN), 1)
              ).astype(jnp.float32)
        qe = jnp.dot(oh, qemb_ref[...], preferred_element_type=jnp.float32)
        fx = jnp.concatenate(
            [ae + qe, emb, px,
             jnp.zeros((BB * s, FP - F), jnp.float32)], axis=1)
    else:
        s = fx_ref.shape[1]
        fx = fx_ref[...].reshape(BB * s, FP)
        px = px_ref[...].reshape(BB * s, P)
    fxb = _bf(fx)
    qkv = (jnp.dot(fxb, qkvw_s[...], preferred_element_type=jnp.float32)
           + qkvb_s[...])
    qkvb16 = _bf(qkv)

    ctx_rows = []
    for i in range(BB):
        rs = slice(i * s, (i + 1) * s)
        ctxs = []
        for h in range(H):
            qsl = slice(h * DHP, (h + 1) * DHP)
            ksl = slice(FP + h * DHP, FP + (h + 1) * DHP)
            vsl = slice(2 * FP + h * DHP, 2 * FP + (h + 1) * DHP)
            sc = lax.dot_general(qkvb16[rs, qsl], qkvb16[rs, ksl],
                                 (((1,), (1,)), ((), ())),
                                 preferred_element_type=jnp.float32)
            e = jnp.exp(sc)
            rinv = 1.0 / jnp.sum(e, axis=-1, keepdims=True)
            ctxs.append(jnp.dot(_bf(e), qkvb16[rs, vsl],
                                preferred_element_type=jnp.float32) * rinv)
        ctx_rows.append(jnp.concatenate(ctxs, axis=-1))
    ctx = _bf(jnp.concatenate(ctx_rows, axis=0))
    att = jnp.dot(ctx, ow_s[...], preferred_element_type=jnp.float32) + ob_s[...]

    fx1 = _ln(fx + att, n1a_s[...], n1b_s[...])

    ff = jnp.maximum(
        jnp.dot(_bf(fx1), f1w_s[...], preferred_element_type=jnp.float32)
        + f1b_s[...], 0.0)
    ff = jnp.dot(_bf(ff), f2w_s[...],
                 preferred_element_type=jnp.float32) + f2b_s[...]
    fx2 = _ln(fx1 + ff, n2a_s[...], n2b_s[...])

    fx2b = _bf(fx2)
    px_new = (jnp.dot(fx2b, pf_s[...], preferred_element_type=jnp.float32)
              + jnp.dot(_bf(px), pp_s[...], preferred_element_type=jnp.float32)
              + rpub[...].reshape(1, P))

    hc = jnp.maximum(
        jnp.dot(fx2b, scw_s[...], preferred_element_type=jnp.float32)
        + scb_s[...], 0.0)
    prod = hc * sc2_s[...]
    sim = (jnp.sum(prod[:, :HLFP], axis=-1, keepdims=True)
           + rs2b[...].reshape(1, 1))
    conf = jax.nn.sigmoid(jnp.sum(prod[:, HLFP:], axis=-1, keepdims=True)
                          + rc2b[...].reshape(1, 1))

    sm_rows = []
    for i in range(BB):
        rs = slice(i * s, (i + 1) * s)
        qrow = qrow_ref[i]
        qcol = qcol_ref[i]
        e = jnp.where(qcol == qrow, jnp.exp(sim[rs]), 1.0)
        px1 = jnp.concatenate(
            [px_new[rs], jnp.ones((s, 1), jnp.float32)], axis=1)
        t = lax.dot_general(_bf(e), _bf(px1), (((0,), (0,)), ((), ())),
                            preferred_element_type=jnp.float32)
        sm_rows.append(t[:, :P] / t[:, P:P + 1])
    smoothed = jnp.concatenate(sm_rows, axis=0)

    if fx_out is not None:
        fx_out[...] = fx2.reshape(BB, s, FP)
    px_out[...] = (conf * px_new
                   + (1.0 - conf) * smoothed).reshape(BB, s, P)


def _body_mid(*args):
    _layer_body(args, write_fx=True)


def _body_last(*args):
    _layer_body(args, write_fx=False)


def _body_first(*args):
    _layer_body(args, write_fx=True, first=True)


def _whole(shape):
    nd = len(shape)
    return pl.BlockSpec(shape, lambda b, _nd=nd: (0,) * _nd)


_SCRATCH = [
    pltpu.VMEM((FP, 3 * FP), jnp.bfloat16),
    pltpu.VMEM((1, 3 * FP), jnp.float32),
    pltpu.VMEM((FP, FP), jnp.bfloat16),
    pltpu.VMEM((1, FP), jnp.float32),
    pltpu.VMEM((FP, DFFP), jnp.bfloat16),
    pltpu.VMEM((1, DFFP), jnp.float32),
    pltpu.VMEM((DFFP, FP), jnp.bfloat16),
    pltpu.VMEM((1, FP), jnp.float32),
    pltpu.VMEM((1, FP), jnp.float32),
    pltpu.VMEM((1, FP), jnp.float32),
    pltpu.VMEM((1, FP), jnp.float32),
    pltpu.VMEM((1, FP), jnp.float32),
    pltpu.VMEM((FP, P), jnp.bfloat16),
    pltpu.VMEM((P, P), jnp.bfloat16),
    pltpu.VMEM((FP, 2 * HLFP), jnp.bfloat16),
    pltpu.VMEM((1, 2 * HLFP), jnp.float32),
    pltpu.VMEM((1, 2 * HLFP), jnp.float32),
]


def _layer_call(ins, qrow, qcol, wts, first=False, last=False,
                interpret=False):
    b, _, s = qrow.shape
    if first:
        x, ae, emb, qemb = ins
        in_specs = [
            pl.BlockSpec((BB, s, MC + 1), lambda i: (i, 0, 0)),
            pl.BlockSpec((BB, s, 128), lambda i: (i, 0, 0)),
            pl.BlockSpec((BB, s, 384), lambda i: (i, 0, 0)),
            _whole(qemb.shape),
        ]
        operands = [x, ae, emb, qemb]
        body0 = _body_first
    else:
        fx, px = ins
        in_specs = [
            pl.BlockSpec((BB, s, FP), lambda i: (i, 0, 0)),
            pl.BlockSpec((BB, s, P), lambda i: (i, 0, 0)),
        ]
        operands = [fx, px]
        body0 = _body_mid
    in_specs += [
        pl.BlockSpec((BB, 1, s), lambda i: (i, 0, 0)),
        pl.BlockSpec((BB, s, 1), lambda i: (i, 0, 0)),
    ] + [_whole(w.shape) for w in wts]
    px_spec = pl.BlockSpec((BB, s, P), lambda i: (i, 0, 0))
    px_shape = jax.ShapeDtypeStruct((b, s, P), jnp.float32)
    if last:
        out_specs, out_shape = px_spec, px_shape
        body = _body_last
    else:
        out_specs = [pl.BlockSpec((BB, s, FP), lambda i: (i, 0, 0)), px_spec]
        out_shape = [jax.ShapeDtypeStruct((b, s, FP), jnp.float32), px_shape]
        body = body0
    out = pl.pallas_call(
        body,
        grid=(b // BB,),
        in_specs=in_specs,
        out_specs=out_specs,
        out_shape=out_shape,
        scratch_shapes=list(_SCRATCH),
        compiler_params=pltpu.CompilerParams(
            dimension_semantics=("arbitrary",),
        ),
        interpret=interpret,
    )(*operands, qrow, qcol, *wts)
    if last:
        return None, out
    return out


def kernel(x, annotators, questions, embeddings, params):
    b, s = annotators.shape
    qidx = questions.astype(jnp.int32)
    ann = annotators.astype(jnp.int32)
    aidx = jnp.where(ann < 0, NA, ann)

    at = jnp.pad(params["a_emb"], ((0, 0), (0, 128 - AED)))
    ae = _sc_gather(at, aidx.reshape(-1)).reshape(b, s, 128)

    qrow = qidx.reshape(b, 1, s)
    qcol = qidx.reshape(b, s, 1)

    fx, px = None, None
    for li, lp in enumerate(params["layers"]):
        wts = [lp[k][:, 0] if k in ("s2w", "c2w") else lp[k]
               for k in RAW_KEYS]
        first = li == 0
        ins = (x, ae, embeddings, params["q_emb"]) if first else (fx, px)
        fx, px = _layer_call(ins, qrow, qcol, wts,
                             first=first, last=(li == NLAYER - 1))
    return px


# speedup vs baseline: 1.3258x; 1.0417x over previous
import functools
import math

import jax
import jax.numpy as jnp
from jax import lax
from jax.experimental import pallas as pl
from jax.experimental.pallas import tpu as pltpu
from jax.experimental.pallas import tpu_sc as plsc

QN = 20
MC = 8
NLAYER = 4
H = 4
NA = 1000
AED = 32
F = AED + MC + 384
P = MC
DFF = 4 * F
DH = F // H

FP = 512
DHP = 128
DFFP = 1792
HLF = F // 2
HLFP = 256

BB = 2


def _sc_gather(a_emb, aidx):
    n = aidx.shape[0]
    d = a_emb.shape[1]
    info = plsc.get_sparse_core_info()
    nc, ns = info.num_cores, info.num_subcores
    nw = nc * ns
    per_w = n // nw
    mesh = plsc.VectorSubcoreMesh(core_axis_name="c", subcore_axis_name="s")

    @functools.partial(
        pl.kernel,
        mesh=mesh,
        out_type=jax.ShapeDtypeStruct((n, d), jnp.float32),
        scratch_types=[
            pltpu.VMEM((per_w,), jnp.int32),
            pltpu.VMEM((per_w, d), jnp.float32),
            pltpu.SemaphoreType.DMA,
        ],
    )
    def gather_k(at_hbm, ai_hbm, ao_hbm, ai_v, ar_v, sem_a):
        wid = lax.axis_index("s") * nc + lax.axis_index("c")
        base = wid * per_w
        pltpu.sync_copy(ai_hbm.at[pl.ds(base, per_w)], ai_v)
        pltpu.async_copy(at_hbm.at[ai_v], ar_v, sem_a).wait()
        pltpu.sync_copy(ar_v, ao_hbm.at[pl.ds(base, per_w)])

    return gather_k(a_emb, aidx)


RAW_KEYS = ("Qw", "Qb", "Kw", "Kb", "Vw", "Vb", "Ow", "Ob",
            "ff1w", "ff1b", "ff2w", "ff2b",
            "n1a", "n1b", "n2a", "n2b",
            "puw", "pub",
            "s1w", "s1b", "s2w", "s2b",
            "c1w", "c1b", "c2w", "c2b")


def _bf(x):
    return x.astype(jnp.bfloat16)


def _pad_rc(w, rows, cols):
    r, c = w.shape
    if cols > c:
        w = jnp.concatenate([w, jnp.zeros((r, cols - c), w.dtype)], axis=1)
    if rows > r:
        w = jnp.concatenate([w, jnp.zeros((rows - r, cols), w.dtype)], axis=0)
    return w


def _perm_cols(w):
    z = jnp.zeros((w.shape[0], DHP - DH), w.dtype)
    parts = []
    for h in range(H):
        parts.append(w[:, h * DH:(h + 1) * DH])
        parts.append(z)
    return jnp.concatenate(parts, axis=1)


def _perm_rows(w):
    z = jnp.zeros((DHP - DH, w.shape[1]), w.dtype)
    parts = []
    for h in range(H):
        parts.append(w[h * DH:(h + 1) * DH, :])
        parts.append(z)
    return jnp.concatenate(parts, axis=0)


def _ln(y, a, b):
    m = jnp.sum(y, axis=-1, keepdims=True) * (1.0 / F)
    ssq = jnp.sum(y * y, axis=-1, keepdims=True)
    var = (ssq - F * m * m) * (1.0 / (F - 1))
    return a * ((y - m) / (jnp.sqrt(var) + 1e-6)) + b


def _prep_scratch(raws, scratch):
    (rQw, rQb, rKw, rKb, rVw, rVb, rOw, rOb,
     rf1w, rf1b, rf2w, rf2b,
     rn1a, rn1b, rn2a, rn2b,
     rpuw, rpub,
     rs1w, rs1b, rs2w, rs2b,
     rc1w, rc1b, rc2w, rc2b) = raws
    (qkvw_s, qkvb_s, ow_s, ob_s,
     f1w_s, f1b_s, f2w_s, f2b_s,
     n1a_s, n1b_s, n2a_s, n2b_s,
     pf_s, pp_s,
     scw_s, scb_s, sc2_s) = scratch

    scale = 1.0 / math.sqrt(DH)
    if True:
        qkvw_s[...] = _bf(jnp.concatenate(
            [_pad_rc(_perm_cols(rQw[...]), FP, FP) * scale,
             _pad_rc(_perm_cols(rKw[...]), FP, FP),
             _pad_rc(_perm_cols(rVw[...]), FP, FP)], axis=1))
        qkvb_s[...] = jnp.concatenate(
            [_perm_cols(rQb[...].reshape(1, F)) * scale,
             _perm_cols(rKb[...].reshape(1, F)),
             _perm_cols(rVb[...].reshape(1, F))], axis=1)
        ow_s[...] = _bf(_pad_rc(_perm_rows(rOw[...]), FP, FP))
        ob_s[...] = _pad_rc(rOb[...].reshape(1, F), 1, FP)
        f1w_s[...] = _bf(_pad_rc(rf1w[...], FP, DFFP))
        f1b_s[...] = _pad_rc(rf1b[...].reshape(1, DFF), 1, DFFP)
        f2w_s[...] = _bf(_pad_rc(rf2w[...], DFFP, FP))
        f2b_s[...] = _pad_rc(rf2b[...].reshape(1, F), 1, FP)
        n1a_s[...] = _pad_rc(rn1a[...].reshape(1, F), 1, FP)
        n1b_s[...] = _pad_rc(rn1b[...].reshape(1, F), 1, FP)
        n2a_s[...] = _pad_rc(rn2a[...].reshape(1, F), 1, FP)
        n2b_s[...] = _pad_rc(rn2b[...].reshape(1, F), 1, FP)
        pf_s[...] = _bf(_pad_rc(rpuw[...][:F, :], FP, P))
        pp_s[...] = _bf(rpuw[...][F:, :])
        scw_s[...] = _bf(jnp.concatenate(
            [_pad_rc(rs1w[...], FP, HLFP),
             _pad_rc(rc1w[...], FP, HLFP)], axis=1))
        scb_s[...] = jnp.concatenate(
            [_pad_rc(rs1b[...].reshape(1, HLF), 1, HLFP),
             _pad_rc(rc1b[...].reshape(1, HLF), 1, HLFP)], axis=1)
        sc2_s[...] = jnp.concatenate(
            [_pad_rc(rs2w[...].reshape(1, HLF), 1, HLFP),
             _pad_rc(rc2w[...].reshape(1, HLF), 1, HLFP)], axis=1)

def _compute_layer(fx, px, qrow_ref, qcol_ref, raws, scratch, s):
    (rQw, rQb, rKw, rKb, rVw, rVb, rOw, rOb,
     rf1w, rf1b, rf2w, rf2b,
     rn1a, rn1b, rn2a, rn2b,
     rpuw, rpub,
     rs1w, rs1b, rs2w, rs2b,
     rc1w, rc1b, rc2w, rc2b) = raws
    (qkvw_s, qkvb_s, ow_s, ob_s,
     f1w_s, f1b_s, f2w_s, f2b_s,
     n1a_s, n1b_s, n2a_s, n2b_s,
     pf_s, pp_s,
     scw_s, scb_s, sc2_s) = scratch

    fxb = _bf(fx)
    qkvb16 = _bf(jnp.dot(fxb, qkvw_s[...],
                         preferred_element_type=jnp.float32)
                 + qkvb_s[...])

    ctx_rows = []
    for i in range(BB):
        rs = slice(i * s, (i + 1) * s)
        ctxs = []
        for h in range(H):
            qsl = slice(h * DHP, (h + 1) * DHP)
            ksl = slice(FP + h * DHP, FP + (h + 1) * DHP)
            vsl = slice(2 * FP + h * DHP, 2 * FP + (h + 1) * DHP)
            sc = lax.dot_general(qkvb16[rs, qsl], qkvb16[rs, ksl],
                                 (((1,), (1,)), ((), ())),
                                 preferred_element_type=jnp.float32)
            e = jnp.exp(sc)
            rinv = 1.0 / jnp.sum(e, axis=-1, keepdims=True)
            ctxs.append(jnp.dot(_bf(e), qkvb16[rs, vsl],
                                preferred_element_type=jnp.float32) * rinv)
        ctx_rows.append(jnp.concatenate(ctxs, axis=-1))
    ctx = _bf(jnp.concatenate(ctx_rows, axis=0))
    att = jnp.dot(ctx, ow_s[...], preferred_element_type=jnp.float32) + ob_s[...]

    fx1 = _ln(fx + att, n1a_s[...], n1b_s[...])

    ff = jnp.maximum(
        jnp.dot(_bf(fx1), f1w_s[...], preferred_element_type=jnp.float32)
        + f1b_s[...], 0.0)
    ff = jnp.dot(_bf(ff), f2w_s[...],
                 preferred_element_type=jnp.float32) + f2b_s[...]
    fx2 = _ln(fx1 + ff, n2a_s[...], n2b_s[...])

    fx2b = _bf(fx2)
    px_new = (jnp.dot(fx2b, pf_s[...], preferred_element_type=jnp.float32)
              + jnp.dot(_bf(px), pp_s[...], preferred_element_type=jnp.float32)
              + rpub[...].reshape(1, P))

    hc = jnp.maximum(
        jnp.dot(fx2b, scw_s[...], preferred_element_type=jnp.float32)
        + scb_s[...], 0.0)
    prod = hc * sc2_s[...]
    sim = (jnp.sum(prod[:, :HLFP], axis=-1, keepdims=True)
           + rs2b[...].reshape(1, 1))
    conf = jax.nn.sigmoid(jnp.sum(prod[:, HLFP:], axis=-1, keepdims=True)
                          + rc2b[...].reshape(1, 1))

    sm_rows = []
    for i in range(BB):
        rs = slice(i * s, (i + 1) * s)
        qrow = qrow_ref[i]
        qcol = qcol_ref[i]
        e = jnp.where(qcol == qrow, jnp.exp(sim[rs]), 1.0)
        px1 = jnp.concatenate(
            [px_new[rs], jnp.ones((s, 1), jnp.float32)], axis=1)
        t = lax.dot_general(_bf(e), _bf(px1), (((0,), (0,)), ((), ())),
                            preferred_element_type=jnp.float32)
        sm_rows.append(t[:, :P] / t[:, P:P + 1])
    smoothed = jnp.concatenate(sm_rows, axis=0)

    return fx2, conf * px_new + (1.0 - conf) * smoothed


NRAW = len(RAW_KEYS)
NSCR = 17


def _pair_body(args, first, write_fx):
    if first:
        (x_ref, ae_ref, emb_ref, qemb_ref, qrow_ref, qcol_ref) = args[:6]
        nd = 6
    else:
        (fx_ref, px_ref, qrow_ref, qcol_ref) = args[:4]
        nd = 4
    raws_a = args[nd:nd + NRAW]
    raws_b = args[nd + NRAW:nd + 2 * NRAW]
    nout = 2 if write_fx else 1
    outs = args[nd + 2 * NRAW:nd + 2 * NRAW + nout]
    scr = args[nd + 2 * NRAW + nout:]
    scr_a, scr_b = scr[:NSCR], scr[NSCR:]

    @pl.when(pl.program_id(0) == 0)
    def _prep():
        _prep_scratch(raws_a, scr_a)
        _prep_scratch(raws_b, scr_b)

    if first:
        s = x_ref.shape[1]
        x9 = x_ref[...].reshape(BB * s, MC + 1)
        px = x9[:, 1:]
        ae = ae_ref[...].reshape(BB * s, 128)[:, :AED]
        emb = emb_ref[...].reshape(BB * s, 384)
        qc = qcol_ref[...].reshape(BB * s, 1)
        oh = (qc == lax.broadcasted_iota(jnp.int32, (1, QN), 1)
              ).astype(jnp.float32)
        qe = jnp.dot(oh, qemb_ref[...], preferred_element_type=jnp.float32)
        fx = jnp.concatenate(
            [ae + qe, emb, px,
             jnp.zeros((BB * s, FP - F), jnp.float32)], axis=1)
    else:
        s = fx_ref.shape[1]
        fx = fx_ref[...].reshape(BB * s, FP)
        px = px_ref[...].reshape(BB * s, P)

    fx, px = _compute_layer(fx, px, qrow_ref, qcol_ref, raws_a, scr_a, s)
    fx, px = _compute_layer(fx, px, qrow_ref, qcol_ref, raws_b, scr_b, s)

    if write_fx:
        fx_out, px_out = outs
        fx_out[...] = fx.reshape(BB, s, FP)
    else:
        (px_out,) = outs
    px_out[...] = px.reshape(BB, s, P)


def _body_first_pair(*args):
    _pair_body(args, first=True, write_fx=True)


def _body_last_pair(*args):
    _pair_body(args, first=False, write_fx=False)


def _whole(shape):
    nd = len(shape)
    return pl.BlockSpec(shape, lambda b, _nd=nd: (0,) * _nd)


_SCRATCH = [
    pltpu.VMEM((FP, 3 * FP), jnp.bfloat16),
    pltpu.VMEM((1, 3 * FP), jnp.float32),
    pltpu.VMEM((FP, FP), jnp.bfloat16),
    pltpu.VMEM((1, FP), jnp.float32),
    pltpu.VMEM((FP, DFFP), jnp.bfloat16),
    pltpu.VMEM((1, DFFP), jnp.float32),
    pltpu.VMEM((DFFP, FP), jnp.bfloat16),
    pltpu.VMEM((1, FP), jnp.float32),
    pltpu.VMEM((1, FP), jnp.float32),
    pltpu.VMEM((1, FP), jnp.float32),
    pltpu.VMEM((1, FP), jnp.float32),
    pltpu.VMEM((1, FP), jnp.float32),
    pltpu.VMEM((FP, P), jnp.bfloat16),
    pltpu.VMEM((P, P), jnp.bfloat16),
    pltpu.VMEM((FP, 2 * HLFP), jnp.bfloat16),
    pltpu.VMEM((1, 2 * HLFP), jnp.float32),
    pltpu.VMEM((1, 2 * HLFP), jnp.float32),
]


def _pair_call(ins, qrow, qcol, wts_a, wts_b, first=False, interpret=False):
    b, _, s = qrow.shape
    if first:
        x, ae, emb, qemb = ins
        in_specs = [
            pl.BlockSpec((BB, s, MC + 1), lambda i: (i, 0, 0)),
            pl.BlockSpec((BB, s, 128), lambda i: (i, 0, 0)),
            pl.BlockSpec((BB, s, 384), lambda i: (i, 0, 0)),
            _whole(qemb.shape),
        ]
        operands = [x, ae, emb, qemb]
        body = _body_first_pair
    else:
        fx, px = ins
        in_specs = [
            pl.BlockSpec((BB, s, FP), lambda i: (i, 0, 0)),
            pl.BlockSpec((BB, s, P), lambda i: (i, 0, 0)),
        ]
        operands = [fx, px]
        body = _body_last_pair
    in_specs += [
        pl.BlockSpec((BB, 1, s), lambda i: (i, 0, 0)),
        pl.BlockSpec((BB, s, 1), lambda i: (i, 0, 0)),
    ] + [_whole(w.shape) for w in wts_a] + [_whole(w.shape) for w in wts_b]
    px_spec = pl.BlockSpec((BB, s, P), lambda i: (i, 0, 0))
    px_shape = jax.ShapeDtypeStruct((b, s, P), jnp.float32)
    if first:
        out_specs = [pl.BlockSpec((BB, s, FP), lambda i: (i, 0, 0)), px_spec]
        out_shape = [jax.ShapeDtypeStruct((b, s, FP), jnp.float32), px_shape]
    else:
        out_specs, out_shape = px_spec, px_shape
    out = pl.pallas_call(
        body,
        grid=(b // BB,),
        in_specs=in_specs,
        out_specs=out_specs,
        out_shape=out_shape,
        scratch_shapes=list(_SCRATCH) + list(_SCRATCH),
        compiler_params=pltpu.CompilerParams(
            dimension_semantics=("arbitrary",),
            vmem_limit_bytes=100 * 1024 * 1024,
        ),
        interpret=interpret,
    )(*operands, qrow, qcol, *wts_a, *wts_b)
    if first:
        return out
    return None, out


def kernel(x, annotators, questions, embeddings, params):
    b, s = annotators.shape
    qidx = questions.astype(jnp.int32)
    ann = annotators.astype(jnp.int32)
    aidx = jnp.where(ann < 0, NA, ann)

    at = jnp.pad(params["a_emb"], ((0, 0), (0, 128 - AED)))
    ae = _sc_gather(at, aidx.reshape(-1)).reshape(b, s, 128)

    qrow = qidx.reshape(b, 1, s)
    qcol = qidx.reshape(b, s, 1)

    def raw(lp):
        return [lp[k][:, 0] if k in ("s2w", "c2w") else lp[k]
                for k in RAW_KEYS]

    lps = params["layers"]
    fx, px = _pair_call((x, ae, embeddings, params["q_emb"]), qrow, qcol,
                        raw(lps[0]), raw(lps[1]), first=True)
    _, px = _pair_call((fx, px), qrow, qcol,
                       raw(lps[2]), raw(lps[3]), first=False)
    return px
